# baseline (speedup 1.0000x reference)
import functools

import jax
import jax.numpy as jnp
import numpy as np
from jax import lax
from jax.experimental import pallas as pl
from jax.experimental.pallas import tpu as pltpu
from jax.experimental.pallas import tpu_sc as plsc

NUM_ROWS = 8
DIM = 100
LANES = 128
CYCLE = 25
ROWS_PER_CYCLE = 32
COMBOS = NUM_ROWS ** 3

_info = plsc.get_sparse_core_info()
_NC, _NS = _info.num_cores, _info.num_subcores
_NW = _NC * _NS

_off = 128 * np.arange(CYCLE)[:, None] + np.arange(LANES)[None, :]
_BASE_ROW = (128 * np.arange(CYCLE)) // 100
_RLOCAL = _off // 100 - _BASE_ROW[:, None]
_COL = _off % 100
_TRIPLE = np.stack(
    [np.arange(COMBOS) // 64, (np.arange(COMBOS) // 8) % 8,
     np.arange(COMBOS) % 8], axis=1)
_ROW_SEL = _TRIPLE[:, _RLOCAL].transpose(1, 0, 2)
_COL_SEL = np.broadcast_to(_COL[:, None, :], _ROW_SEL.shape)


def _make_sc_gather(NB: int, C: int):
    per_w = NB // _NW
    n_chunks = per_w // C
    assert n_chunks % 2 == 0 and n_chunks * C == per_w
    mesh = plsc.VectorSubcoreMesh(core_axis_name="c", subcore_axis_name="s")

    @functools.partial(
        pl.kernel,
        mesh=mesh,
        out_type=jax.ShapeDtypeStruct((NB, LANES), jnp.float32),
        scratch_types=[
            pltpu.VMEM((per_w,), jnp.int32),
            pltpu.VMEM((C, LANES), jnp.float32),
            pltpu.VMEM((C, LANES), jnp.float32),
            pltpu.SemaphoreType.DMA,
            pltpu.SemaphoreType.DMA,
            pltpu.SemaphoreType.DMA,
            pltpu.SemaphoreType.DMA,
        ],
    )
    def k(idx_hbm, blocks_hbm, out_hbm, idx_v, rows0, rows1,
          sg0, sg1, sw0, sw1):
        wid = lax.axis_index("s") * _NC + lax.axis_index("c")
        w0 = wid * per_w
        rows = (rows0, rows1)
        sg = (sg0, sg1)
        sw = (sw0, sw1)

        pltpu.sync_copy(idx_hbm.at[pl.ds(w0, per_w)], idx_v)

        def gather_start(g, b):
            pltpu.async_copy(
                blocks_hbm.at[idx_v.at[pl.ds(g * C, C)]], rows[b], sg[b])

        def gather_wait(g, b):
            pltpu.make_async_copy(
                blocks_hbm.at[idx_v.at[pl.ds(g * C, C)]], rows[b], sg[b]
            ).wait()

        def wout_start(g, b):
            pltpu.async_copy(rows[b], out_hbm.at[pl.ds(w0 + g * C, C)], sw[b])

        def wout_wait(g, b):
            pltpu.make_async_copy(
                rows[b], out_hbm.at[pl.ds(w0 + g * C, C)], sw[b]).wait()

        gather_start(0, 0)
        gather_start(1, 1)

        def body(i, carry):
            for b in (0, 1):
                g = 2 * i + b
                gather_wait(g, b)
                wout_start(g, b)
                @pl.when(i < n_chunks // 2 - 1)
                def _():
                    wout_wait(g, b)
                    gather_start(g + 2, b)
            return carry

        lax.fori_loop(0, n_chunks // 2, body, 0)
        wout_wait(n_chunks - 2, 0)
        wout_wait(n_chunks - 1, 1)

    return k


def kernel(input, table):
    idx = input.reshape(-1).astype(jnp.int32)
    n_rows = idx.shape[0]
    n_blocks = n_rows * DIM // LANES

    blocks = table[_ROW_SEL, _COL_SEL].reshape(CYCLE * COMBOS, LANES)

    j = jnp.arange(n_blocks, dtype=jnp.int32)
    p = j % CYCLE
    base = (j // CYCLE) * ROWS_PER_CYCLE + jnp.asarray(
        _BASE_ROW, dtype=jnp.int32)[p]
    idx_pad = jnp.concatenate([idx, jnp.zeros((2,), jnp.int32)])
    block_id = (p * COMBOS + idx_pad[base] * 64 + idx_pad[base + 1] * 8
                + idx_pad[base + 2])

    out = _make_sc_gather(n_blocks, 400)(block_id, blocks)
    return out.reshape(input.shape + (DIM,))


# speedup vs baseline: 14.4291x; 14.4291x over previous
import functools

import jax
import jax.numpy as jnp
from jax import lax
from jax.experimental import pallas as pl
from jax.experimental.pallas import tpu as pltpu
from jax.experimental.pallas import tpu_sc as plsc

NUM_ROWS = 8
DIM = 100
DIM_PAD = 128

_info = plsc.get_sparse_core_info()
_NC, _NS = _info.num_cores, _info.num_subcores
_NW = _NC * _NS


def _make_sc_gather(B: int, C: int):
    per_w = B // _NW
    n_chunks = per_w // C
    assert n_chunks % 2 == 0 and n_chunks * C == per_w
    mesh = plsc.VectorSubcoreMesh(core_axis_name="c", subcore_axis_name="s")

    @functools.partial(
        pl.kernel,
        mesh=mesh,
        out_type=jax.ShapeDtypeStruct((B, DIM_PAD), jnp.float32),
        scratch_types=[
            pltpu.VMEM((per_w,), jnp.int32),
            pltpu.VMEM((C, DIM_PAD), jnp.float32),
            pltpu.VMEM((C, DIM_PAD), jnp.float32),
            pltpu.SemaphoreType.DMA,
            pltpu.SemaphoreType.DMA,
            pltpu.SemaphoreType.DMA,
            pltpu.SemaphoreType.DMA,
        ],
    )
    def k(idx_hbm, table_hbm, out_hbm, idx_v, rows0, rows1,
          sg0, sg1, sw0, sw1):
        wid = lax.axis_index("s") * _NC + lax.axis_index("c")
        w0 = wid * per_w
        rows = (rows0, rows1)
        sg = (sg0, sg1)
        sw = (sw0, sw1)

        pltpu.sync_copy(idx_hbm.at[pl.ds(w0, per_w)], idx_v)

        def gather_start(g, b):
            pltpu.async_copy(
                table_hbm.at[idx_v.at[pl.ds(g * C, C)]], rows[b], sg[b])

        def gather_wait(g, b):
            pltpu.make_async_copy(
                table_hbm.at[idx_v.at[pl.ds(g * C, C)]], rows[b], sg[b]
            ).wait()

        def wout_start(g, b):
            pltpu.async_copy(rows[b], out_hbm.at[pl.ds(w0 + g * C, C)], sw[b])

        def wout_wait(g, b):
            pltpu.make_async_copy(
                rows[b], out_hbm.at[pl.ds(w0 + g * C, C)], sw[b]).wait()

        gather_start(0, 0)
        gather_start(1, 1)

        def body(i, carry):
            for b in (0, 1):
                g = 2 * i + b
                gather_wait(g, b)
                wout_start(g, b)
                @pl.when(i < n_chunks // 2 - 1)
                def _():
                    wout_wait(g, b)
                    gather_start(g + 2, b)
            return carry

        lax.fori_loop(0, n_chunks // 2, body, 0)
        wout_wait(n_chunks - 2, 0)
        wout_wait(n_chunks - 1, 1)

    return k


REPLICAS = 2048


def kernel(input, table):
    idx = input.reshape(-1).astype(jnp.int32)
    table_pad = jnp.pad(table, ((0, 0), (0, DIM_PAD - DIM)))
    table_rep = jnp.tile(table_pad, (REPLICAS, 1))
    idx = idx + NUM_ROWS * (
        jnp.arange(idx.shape[0], dtype=jnp.int32) % REPLICAS)
    out = _make_sc_gather(idx.shape[0], 320)(idx, table_rep)
    return out[:, :DIM].reshape(input.shape + (DIM,))
